# Initial kernel scaffold; baseline (speedup 1.0000x reference)
import jax
import jax.numpy as jnp
from jax.experimental import pallas as pl


def kernel(inputs, targets):
    raise NotImplementedError("write your pallas kernel here")


# speedup vs baseline: 81.0087x; 81.0087x over previous
import functools

import jax
import jax.numpy as jnp
from jax import lax
from jax.experimental import pallas as pl
from jax.experimental.pallas import tpu as pltpu
from jax.experimental.pallas import tpu_sc as plsc

_N = 16777216
_NC = 2
_NS = 16
_NW = _NC * _NS
_L = 16
_CHUNK = 16384
_PER_W = _N // _NW
_NCHUNK = _PER_W // _CHUNK
_VECS = _CHUNK // _L


def _fnr_body(x_hbm, t_hbm, out_hbm, xbuf, tbuf, obuf, sx0, sx1, st0, st1):
    wid = lax.axis_index("s") * _NC + lax.axis_index("c")
    first = wid * _NCHUNK
    sems_x = (sx0, sx1)
    sems_t = (st0, st1)

    def issue(ci, b):
        pltpu.make_async_copy(x_hbm.at[ci], xbuf.at[b], sems_x[b]).start()
        pltpu.make_async_copy(t_hbm.at[ci], tbuf.at[b], sems_t[b]).start()

    def wait_slot(b):
        pltpu.make_async_copy(x_hbm.at[0], xbuf.at[b], sems_x[b]).wait()
        pltpu.make_async_copy(t_hbm.at[0], tbuf.at[b], sems_t[b]).wait()

    def chunk_acc(b, carry):
        def body(j, carry):
            fn, pos = carry
            x = xbuf[b, pl.ds(j * _L, _L)]
            t = tbuf[b, pl.ds(j * _L, _L)]
            fn = fn + jnp.where(x < 0.5, t, 0)
            pos = pos + t
            return fn, pos

        return lax.fori_loop(0, _VECS, body, carry, unroll=8)

    issue(first, 0)
    zero = jnp.zeros((_L,), jnp.int32)

    def outer(g, carry):
        for b in range(2):
            i = g * 2 + b

            @pl.when(i + 1 < _NCHUNK)
            def _():
                issue(first + i + 1, 1 - b)

            wait_slot(b)
            carry = chunk_acc(b, carry)
        return carry

    acc_fn, acc_pos = lax.fori_loop(0, _NCHUNK // 2, outer, (zero, zero))

    obuf[0, :] = acc_fn
    obuf[1, :] = acc_pos
    pltpu.sync_copy(obuf, out_hbm.at[wid])


_fnr = functools.partial(
    pl.kernel,
    out_type=jax.ShapeDtypeStruct((_NW, 2, _L), jnp.int32),
    mesh=plsc.VectorSubcoreMesh(core_axis_name="c", subcore_axis_name="s"),
    scratch_types=[
        pltpu.VMEM((2, _CHUNK), jnp.float32),
        pltpu.VMEM((2, _CHUNK), jnp.int32),
        pltpu.VMEM((2, _L), jnp.int32),
        pltpu.SemaphoreType.DMA,
        pltpu.SemaphoreType.DMA,
        pltpu.SemaphoreType.DMA,
        pltpu.SemaphoreType.DMA,
    ],
)(_fnr_body)


@jax.jit
def kernel(inputs, targets):
    x2d = inputs.reshape(_NW * _NCHUNK, _CHUNK)
    t2d = targets.reshape(_NW * _NCHUNK, _CHUNK)
    parts = _fnr(x2d, t2d)
    sums = parts.sum(axis=(0, 2))
    fn = sums[0].astype(jnp.float32)
    pos = sums[1].astype(jnp.float32)
    return fn / jnp.maximum(pos, 1.0)


# speedup vs baseline: 191.4135x; 2.3629x over previous
import functools

import jax
import jax.numpy as jnp
from jax import lax
from jax.experimental import pallas as pl
from jax.experimental.pallas import tpu as pltpu
from jax.experimental.pallas import tpu_sc as plsc

_N = 16777216
_NC = 2
_NS = 16
_NW = _NC * _NS
_L = 16
_CHUNK = 16384
_PER_W = _N // _NW
_NCHUNK = _PER_W // _CHUNK
_VECS = _CHUNK // _L


def _fnr_body(x_hbm, t_hbm, out_hbm, xbuf, tbuf, obuf, sx0, sx1, st0, st1):
    wid = lax.axis_index("s") * _NC + lax.axis_index("c")
    base = wid * _PER_W
    sems_x = (sx0, sx1)
    sems_t = (st0, st1)

    def issue(i, b):
        off = pl.multiple_of(base + i * _CHUNK, _CHUNK)
        pltpu.make_async_copy(x_hbm.at[pl.ds(off, _CHUNK)], xbuf.at[b],
                              sems_x[b]).start()
        pltpu.make_async_copy(t_hbm.at[pl.ds(off, _CHUNK)], tbuf.at[b],
                              sems_t[b]).start()

    def wait_slot(b):
        pltpu.make_async_copy(x_hbm.at[pl.ds(0, _CHUNK)], xbuf.at[b],
                              sems_x[b]).wait()
        pltpu.make_async_copy(t_hbm.at[pl.ds(0, _CHUNK)], tbuf.at[b],
                              sems_t[b]).wait()

    def chunk_acc(b, carry):
        def body(j, carry):
            fn, pos = carry
            x = xbuf[b, pl.ds(j * _L, _L)]
            t = tbuf[b, pl.ds(j * _L, _L)]
            fn = fn + jnp.where(x < 0.5, t, 0)
            pos = pos + t
            return fn, pos

        return lax.fori_loop(0, _VECS, body, carry, unroll=8)

    issue(0, 0)
    zero = jnp.zeros((_L,), jnp.int32)

    def outer(g, carry):
        for b in range(2):
            i = g * 2 + b

            @pl.when(i + 1 < _NCHUNK)
            def _():
                issue(i + 1, 1 - b)

            wait_slot(b)
            carry = chunk_acc(b, carry)
        return carry

    acc_fn, acc_pos = lax.fori_loop(0, _NCHUNK // 2, outer, (zero, zero))

    obuf[0, :] = acc_fn
    obuf[1, :] = acc_pos
    pltpu.sync_copy(obuf, out_hbm.at[wid])


_fnr = functools.partial(
    pl.kernel,
    out_type=jax.ShapeDtypeStruct((_NW, 2, _L), jnp.int32),
    mesh=plsc.VectorSubcoreMesh(core_axis_name="c", subcore_axis_name="s"),
    scratch_types=[
        pltpu.VMEM((2, _CHUNK), jnp.float32),
        pltpu.VMEM((2, _CHUNK), jnp.int32),
        pltpu.VMEM((2, _L), jnp.int32),
        pltpu.SemaphoreType.DMA,
        pltpu.SemaphoreType.DMA,
        pltpu.SemaphoreType.DMA,
        pltpu.SemaphoreType.DMA,
    ],
)(_fnr_body)


@jax.jit
def kernel(inputs, targets):
    parts = _fnr(inputs, targets)
    sums = parts.sum(axis=(0, 2))
    fn = sums[0].astype(jnp.float32)
    pos = sums[1].astype(jnp.float32)
    return fn / jnp.maximum(pos, 1.0)
